# baseline (speedup 1.0000x reference)
import functools

import jax
import jax.numpy as jnp
from jax import lax
from jax.experimental import pallas as pl
from jax.experimental.pallas import tpu as pltpu
from jax.experimental.pallas import tpu_sc as plsc

B, G, T, E, H = 8, 64, 512, 128, 1024
NW = 32
PP = (B * G) // NW
TE = T * E
HT = T // 2
HTE = HT * E
EB = E // 16


def _mm_body(h_ref, w_ref, o_ref):
    o_ref[...] = jnp.dot(h_ref[...], w_ref[...],
                         preferred_element_type=jnp.float32)


_tc_matmul = pl.pallas_call(
    _mm_body,
    out_shape=jax.ShapeDtypeStruct((B, E), jnp.float32),
)


def _sc_body(emb_hbm, inter_hbm, atop_hbm, alpha_hbm, cpart_hbm,
             h0, h1, h2, inter_v, atop_v, beta_v, c_v,
             s0, s1, s2, s_alpha):
    wid = lax.axis_index("s") * 2 + lax.axis_index("c")
    b = wid // (NW // B)
    pltpu.sync_copy(inter_hbm.at[b], inter_v)
    pltpu.sync_copy(atop_hbm.at[pl.ds(wid * PP, PP)], atop_v)

    zero16 = jnp.zeros((16,), jnp.float32)
    iota16 = lax.iota(jnp.int32, 16)
    for eb in range(EB):
        c_v[pl.ds(eb * 16, 16)] = zero16
    ivs = [inter_v[pl.ds(eb * 16, 16)] for eb in range(EB)]
    atop_reg = atop_v[...]

    def dma_start(p, half, buf, sem):
        pltpu.async_copy(
            emb_hbm.at[pl.ds(p * TE + half * HTE, HTE)], buf, sem)

    def dma_wait(p, half, buf, sem):
        pltpu.make_async_copy(
            emb_hbm.at[pl.ds(p * TE + half * HTE, HTE)], buf, sem).wait()

    def fused_half(buf, beta_off, carry):
        @plsc.parallel_loop(0, HT // 16, carry=tuple(carry))
        def out_carry(tb, carry):
            mv, sv, *w = carry
            base0 = tb * (16 * E)
            betav = zero16
            for g in range(8):
                r0 = base0 + (2 * g) * E
                r1 = r0 + E
                xa = [buf[pl.ds(r0 + eb * 16, 16)] for eb in range(EB)]
                xb = [buf[pl.ds(r1 + eb * 16, 16)] for eb in range(EB)]
                da = [xa[eb] * ivs[eb] for eb in range(EB)]
                db = [xb[eb] * ivs[eb] for eb in range(EB)]
                sa = ((da[0] + da[1]) + (da[2] + da[3])) + \
                     ((da[4] + da[5]) + (da[6] + da[7]))
                sb = ((db[0] + db[1]) + (db[2] + db[3])) + \
                     ((db[4] + db[5]) + (db[6] + db[7]))
                b0 = jnp.full((16,), jnp.sum(sa), jnp.float32)
                b1 = jnp.full((16,), jnp.sum(sb), jnp.float32)
                betav = jnp.where(iota16 == 2 * g, b0, betav)
                betav = jnp.where(iota16 == 2 * g + 1, b1, betav)
                mn = jnp.maximum(mv, jnp.maximum(b0, b1))
                corr = jnp.exp(mv - mn)
                e0 = jnp.exp(b0 - mn)
                e1 = jnp.exp(b1 - mn)
                sv = sv * corr + (e0 + e1)
                w = [w[eb] * corr + (xa[eb] * e0 + xb[eb] * e1)
                     for eb in range(EB)]
                mv = mn
            beta_v[pl.ds(beta_off + tb * 16, 16)] = betav
            return (mv, sv, *w)

        return out_carry

    def do_pair(i, lo, hi, nxt, s_lo, s_hi, s_nxt):
        p = wid * PP + i
        dma_start(p, 1, hi, s_hi)
        dma_wait(p, 0, lo, s_lo)
        neg_inf = jnp.full((16,), -jnp.inf, jnp.float32)
        carry = fused_half(lo, 0, (neg_inf, zero16) + (zero16,) * EB)
        dma_wait(p, 1, hi, s_hi)
        pn = jnp.minimum(p + 1, B * G - 1)
        dma_start(pn, 0, nxt, s_nxt)
        mv, sv, *w = fused_half(hi, HT, carry)
        atop_i = jnp.sum(jnp.where(iota16 == i, atop_reg, 0.0))
        scs = jnp.full((16,), atop_i, jnp.float32) / sv

        @plsc.parallel_loop(0, T // 16, unroll=2)
        def _alpha(j):
            beta_v[pl.ds(j * 16, 16)] = \
                jnp.exp(beta_v[pl.ds(j * 16, 16)] - mv) * scs

        pltpu.async_copy(beta_v, alpha_hbm.at[p], s_alpha)
        for eb in range(EB):
            c_v[pl.ds(eb * 16, 16)] = c_v[pl.ds(eb * 16, 16)] + w[eb] * scs
        pltpu.make_async_copy(beta_v, alpha_hbm.at[p], s_alpha).wait()

    dma_start(wid * PP, 0, h0, s0)
    do_pair(0, h0, h1, h2, s0, s1, s2)

    def k_body(k, _):
        i1 = 1 + 3 * k
        do_pair(i1, h2, h0, h1, s2, s0, s1)
        do_pair(i1 + 1, h1, h2, h0, s1, s2, s0)
        do_pair(i1 + 2, h0, h1, h2, s0, s1, s2)
        return 0

    lax.fori_loop(0, (PP - 1) // 3, k_body, 0)
    pltpu.make_async_copy(
        emb_hbm.at[pl.ds(0, HTE)], h2, s2).wait()
    pltpu.sync_copy(c_v, cpart_hbm.at[wid])


_sc_call = functools.partial(
    pl.kernel,
    mesh=plsc.VectorSubcoreMesh(core_axis_name="c", subcore_axis_name="s"),
    compiler_params=pltpu.CompilerParams(needs_layout_passes=False),
    out_type=(
        jax.ShapeDtypeStruct((B * G, T), jnp.float32),
        jax.ShapeDtypeStruct((NW, E), jnp.float32),
    ),
    scratch_types=[
        pltpu.VMEM((HTE,), jnp.float32),
        pltpu.VMEM((HTE,), jnp.float32),
        pltpu.VMEM((HTE,), jnp.float32),
        pltpu.VMEM((E,), jnp.float32),
        pltpu.VMEM((PP,), jnp.float32),
        pltpu.VMEM((T,), jnp.float32),
        pltpu.VMEM((E,), jnp.float32),
        pltpu.SemaphoreType.DMA,
        pltpu.SemaphoreType.DMA,
        pltpu.SemaphoreType.DMA,
        pltpu.SemaphoreType.DMA,
    ],
)(_sc_body)


def kernel(decoder_hidden_state, alpha_graph_attention_top, all_embeddings, W):
    inter = _tc_matmul(decoder_hidden_state, W)
    emb_flat = all_embeddings.reshape(-1)
    atop_flat = alpha_graph_attention_top.reshape(-1)
    alpha_flat, c_part = _sc_call(emb_flat, inter, atop_flat)
    c = c_part.reshape(B, NW // B, E).sum(axis=1)
    alpha = alpha_flat.reshape(B, G, T)
    return (c, alpha)


# speedup vs baseline: 1.0072x; 1.0072x over previous
import functools

import jax
import jax.numpy as jnp
from jax import lax
from jax.experimental import pallas as pl
from jax.experimental.pallas import tpu as pltpu
from jax.experimental.pallas import tpu_sc as plsc

B, G, T, E, H = 8, 64, 512, 128, 1024
NW = 32
PP = (B * G) // NW
TE = T * E
HT = T // 2
HTE = HT * E
EB = E // 16


def _mm_body(h_ref, w_ref, o_ref):
    o_ref[...] = jnp.dot(h_ref[...], w_ref[...],
                         preferred_element_type=jnp.float32)


_tc_matmul = pl.pallas_call(
    _mm_body,
    out_shape=jax.ShapeDtypeStruct((B, E), jnp.float32),
)


def _sc_body(emb_hbm, inter_hbm, atop_hbm, alpha_hbm, cpart_hbm,
             h0, h1, h2, inter_v, atop_v, beta_v, c_v,
             s0, s1, s2, s_alpha):
    wid = lax.axis_index("s") * 2 + lax.axis_index("c")
    b = wid // (NW // B)
    pltpu.sync_copy(inter_hbm.at[b], inter_v)
    pltpu.sync_copy(atop_hbm.at[pl.ds(wid * PP, PP)], atop_v)

    zero16 = jnp.zeros((16,), jnp.float32)
    iota16 = lax.iota(jnp.int32, 16)
    for eb in range(EB):
        c_v[pl.ds(eb * 16, 16)] = zero16
    ivs = [inter_v[pl.ds(eb * 16, 16)] for eb in range(EB)]
    atop_reg = atop_v[...]

    QTE = HTE // 2

    def dma_start(p, half, buf, sem):
        base = p * TE + half * HTE
        pltpu.async_copy(
            emb_hbm.at[pl.ds(base, QTE)], buf.at[pl.ds(0, QTE)], sem)
        pltpu.async_copy(
            emb_hbm.at[pl.ds(base + QTE, QTE)], buf.at[pl.ds(QTE, QTE)], sem)

    def dma_wait(p, half, buf, sem):
        for q in range(2):
            pltpu.make_async_copy(
                emb_hbm.at[pl.ds(p * TE + half * HTE + q * QTE, QTE)],
                buf.at[pl.ds(q * QTE, QTE)], sem).wait()

    def fused_half(buf, beta_off, carry):
        @plsc.parallel_loop(0, HT // 16, carry=tuple(carry))
        def out_carry(tb, carry):
            mv, sv, *w = carry
            base0 = tb * (16 * E)
            betav = zero16
            for g in range(8):
                r0 = base0 + (2 * g) * E
                r1 = r0 + E
                xa = [buf[pl.ds(r0 + eb * 16, 16)] for eb in range(EB)]
                xb = [buf[pl.ds(r1 + eb * 16, 16)] for eb in range(EB)]
                da = [xa[eb] * ivs[eb] for eb in range(EB)]
                db = [xb[eb] * ivs[eb] for eb in range(EB)]
                sa = ((da[0] + da[1]) + (da[2] + da[3])) + \
                     ((da[4] + da[5]) + (da[6] + da[7]))
                sb = ((db[0] + db[1]) + (db[2] + db[3])) + \
                     ((db[4] + db[5]) + (db[6] + db[7]))
                b0 = jnp.full((16,), jnp.sum(sa), jnp.float32)
                b1 = jnp.full((16,), jnp.sum(sb), jnp.float32)
                betav = jnp.where(iota16 == 2 * g, b0, betav)
                betav = jnp.where(iota16 == 2 * g + 1, b1, betav)
                mn = jnp.maximum(mv, jnp.maximum(b0, b1))
                corr = jnp.exp(mv - mn)
                e0 = jnp.exp(b0 - mn)
                e1 = jnp.exp(b1 - mn)
                sv = sv * corr + (e0 + e1)
                w = [w[eb] * corr + (xa[eb] * e0 + xb[eb] * e1)
                     for eb in range(EB)]
                mv = mn
            beta_v[pl.ds(beta_off + tb * 16, 16)] = betav
            return (mv, sv, *w)

        return out_carry

    def do_pair(i, lo, hi, nxt, s_lo, s_hi, s_nxt):
        p = wid * PP + i
        dma_start(p, 1, hi, s_hi)
        dma_wait(p, 0, lo, s_lo)
        neg_inf = jnp.full((16,), -jnp.inf, jnp.float32)
        carry = fused_half(lo, 0, (neg_inf, zero16) + (zero16,) * EB)
        dma_wait(p, 1, hi, s_hi)
        pn = jnp.minimum(p + 1, B * G - 1)
        dma_start(pn, 0, nxt, s_nxt)
        mv, sv, *w = fused_half(hi, HT, carry)
        atop_i = jnp.sum(jnp.where(iota16 == i, atop_reg, 0.0))
        scs = jnp.full((16,), atop_i, jnp.float32) / sv

        @plsc.parallel_loop(0, T // 16, unroll=2)
        def _alpha(j):
            beta_v[pl.ds(j * 16, 16)] = \
                jnp.exp(beta_v[pl.ds(j * 16, 16)] - mv) * scs

        pltpu.async_copy(beta_v, alpha_hbm.at[p], s_alpha)
        for eb in range(EB):
            c_v[pl.ds(eb * 16, 16)] = c_v[pl.ds(eb * 16, 16)] + w[eb] * scs
        pltpu.make_async_copy(beta_v, alpha_hbm.at[p], s_alpha).wait()

    dma_start(wid * PP, 0, h0, s0)
    do_pair(0, h0, h1, h2, s0, s1, s2)

    def k_body(k, _):
        i1 = 1 + 3 * k
        do_pair(i1, h2, h0, h1, s2, s0, s1)
        do_pair(i1 + 1, h1, h2, h0, s1, s2, s0)
        do_pair(i1 + 2, h0, h1, h2, s0, s1, s2)
        return 0

    lax.fori_loop(0, (PP - 1) // 3, k_body, 0)
    pltpu.make_async_copy(
        emb_hbm.at[pl.ds(0, HTE)], h2, s2).wait()
    pltpu.sync_copy(c_v, cpart_hbm.at[wid])


_sc_call = functools.partial(
    pl.kernel,
    mesh=plsc.VectorSubcoreMesh(core_axis_name="c", subcore_axis_name="s"),
    compiler_params=pltpu.CompilerParams(needs_layout_passes=False),
    out_type=(
        jax.ShapeDtypeStruct((B * G, T), jnp.float32),
        jax.ShapeDtypeStruct((NW, E), jnp.float32),
    ),
    scratch_types=[
        pltpu.VMEM((HTE,), jnp.float32),
        pltpu.VMEM((HTE,), jnp.float32),
        pltpu.VMEM((HTE,), jnp.float32),
        pltpu.VMEM((E,), jnp.float32),
        pltpu.VMEM((PP,), jnp.float32),
        pltpu.VMEM((T,), jnp.float32),
        pltpu.VMEM((E,), jnp.float32),
        pltpu.SemaphoreType.DMA,
        pltpu.SemaphoreType.DMA,
        pltpu.SemaphoreType.DMA,
        pltpu.SemaphoreType.DMA,
    ],
)(_sc_body)


def kernel(decoder_hidden_state, alpha_graph_attention_top, all_embeddings, W):
    inter = _tc_matmul(decoder_hidden_state, W)
    emb_flat = all_embeddings.reshape(-1)
    atop_flat = alpha_graph_attention_top.reshape(-1)
    alpha_flat, c_part = _sc_call(emb_flat, inter, atop_flat)
    c = c_part.reshape(B, NW // B, E).sum(axis=1)
    alpha = alpha_flat.reshape(B, G, T)
    return (c, alpha)


# speedup vs baseline: 1.0125x; 1.0052x over previous
import functools

import jax
import jax.numpy as jnp
from jax import lax
from jax.experimental import pallas as pl
from jax.experimental.pallas import tpu as pltpu
from jax.experimental.pallas import tpu_sc as plsc

B, G, T, E, H = 8, 64, 512, 128, 1024
NW = 32
PP = (B * G) // NW
TE = T * E
HT = T // 2
HTE = HT * E
EB = E // 16


def _mm_body(h_ref, w_ref, o_ref):
    o_ref[...] = jnp.dot(h_ref[...], w_ref[...],
                         preferred_element_type=jnp.float32)


_tc_matmul = pl.pallas_call(
    _mm_body,
    out_shape=jax.ShapeDtypeStruct((B, E), jnp.float32),
)


def _sc_body(emb_hbm, inter_hbm, atop_hbm, alpha_hbm, cpart_hbm,
             h0, h1, h2, inter_v, atop_v, beta_v, c_v,
             s0, s1, s2, s_alpha):
    wid = lax.axis_index("s") * 2 + lax.axis_index("c")
    b = wid // (NW // B)
    pltpu.sync_copy(inter_hbm.at[b], inter_v)
    pltpu.sync_copy(atop_hbm.at[pl.ds(wid * PP, PP)], atop_v)

    zero16 = jnp.zeros((16,), jnp.float32)
    iota16 = lax.iota(jnp.int32, 16)
    for eb in range(EB):
        c_v[pl.ds(eb * 16, 16)] = zero16
    ivs = [inter_v[pl.ds(eb * 16, 16)] for eb in range(EB)]
    atop_reg = atop_v[...]

    def dma_start(p, half, buf, sem):
        pltpu.async_copy(
            emb_hbm.at[pl.ds(p * TE + half * HTE, HTE)], buf, sem)

    def dma_wait(p, half, buf, sem):
        pltpu.make_async_copy(
            emb_hbm.at[pl.ds(p * TE + half * HTE, HTE)], buf, sem).wait()

    def stage1_half(buf, beta_off):
        @plsc.parallel_loop(0, HT // 16, unroll=2)
        def tb_body(tb):
            base0 = tb * (16 * E)
            betav = zero16
            for lane in range(16):
                roff = base0 + lane * E
                m = [buf[pl.ds(roff + eb * 16, 16)] * ivs[eb]
                     for eb in range(EB)]
                r = ((m[0] + m[1]) + (m[2] + m[3])) + \
                    ((m[4] + m[5]) + (m[6] + m[7]))
                betav = jnp.where(iota16 == lane, jnp.sum(r), betav)
            beta_v[pl.ds(beta_off + tb * 16, 16)] = betav

    def softmax_scale(i):
        @plsc.parallel_loop(0, T // 16, unroll=2,
                            carry=jnp.full((16,), -jnp.inf, jnp.float32))
        def mv(j, m):
            return jnp.maximum(m, beta_v[pl.ds(j * 16, 16)])

        ms = jnp.full((16,), jnp.max(mv), jnp.float32)

        @plsc.parallel_loop(0, T // 16, unroll=2, carry=zero16)
        def sv(j, s):
            ev = jnp.exp(beta_v[pl.ds(j * 16, 16)] - ms)
            beta_v[pl.ds(j * 16, 16)] = ev
            return s + ev

        atop_i = jnp.sum(jnp.where(iota16 == i, atop_reg, 0.0))
        scs = (jnp.full((16,), atop_i, jnp.float32)
               / jnp.full((16,), jnp.sum(sv), jnp.float32))

        @plsc.parallel_loop(0, T // 16, unroll=2)
        def _scale(j):
            beta_v[pl.ds(j * 16, 16)] = beta_v[pl.ds(j * 16, 16)] * scs

    def stage2_half(buf, beta_off, accs):
        @plsc.parallel_loop(0, HT // 16, unroll=2, carry=tuple(accs))
        def out_accs(tb, accs):
            av16 = beta_v[pl.ds(beta_off + tb * 16, 16)]
            base = tb * (16 * E)
            accs = list(accs)
            for lane in range(16):
                av = jnp.full((16,), av16[lane], jnp.float32)
                toff = base + lane * E
                for eb in range(EB):
                    accs[eb] = accs[eb] + buf[pl.ds(toff + eb * 16, 16)] * av
            return tuple(accs)

        return out_accs

    def do_pair(i, lo, hi, nxt, s_lo, s_hi, s_nxt):
        p = wid * PP + i
        dma_start(p, 1, hi, s_hi)
        dma_wait(p, 0, lo, s_lo)
        stage1_half(lo, 0)
        dma_wait(p, 1, hi, s_hi)
        stage1_half(hi, HT)
        softmax_scale(i)
        pltpu.async_copy(beta_v, alpha_hbm.at[p], s_alpha)
        pn = jnp.minimum(p + 1, B * G - 1)
        dma_start(pn, 0, nxt, s_nxt)
        accs = stage2_half(lo, 0, (zero16,) * EB)
        accs = stage2_half(hi, HT, accs)
        for eb in range(EB):
            c_v[pl.ds(eb * 16, 16)] = c_v[pl.ds(eb * 16, 16)] + accs[eb]
        pltpu.make_async_copy(beta_v, alpha_hbm.at[p], s_alpha).wait()

    dma_start(wid * PP, 0, h0, s0)
    do_pair(0, h0, h1, h2, s0, s1, s2)

    def k_body(k, _):
        i1 = 1 + 3 * k
        do_pair(i1, h2, h0, h1, s2, s0, s1)
        do_pair(i1 + 1, h1, h2, h0, s1, s2, s0)
        do_pair(i1 + 2, h0, h1, h2, s0, s1, s2)
        return 0

    lax.fori_loop(0, (PP - 1) // 3, k_body, 0)
    pltpu.make_async_copy(
        emb_hbm.at[pl.ds(0, HTE)], h2, s2).wait()
    pltpu.sync_copy(c_v, cpart_hbm.at[wid])


_sc_call = functools.partial(
    pl.kernel,
    mesh=plsc.VectorSubcoreMesh(core_axis_name="c", subcore_axis_name="s"),
    compiler_params=pltpu.CompilerParams(needs_layout_passes=False),
    out_type=(
        jax.ShapeDtypeStruct((B * G, T), jnp.float32),
        jax.ShapeDtypeStruct((NW, E), jnp.float32),
    ),
    scratch_types=[
        pltpu.VMEM((HTE,), jnp.float32),
        pltpu.VMEM((HTE,), jnp.float32),
        pltpu.VMEM((HTE,), jnp.float32),
        pltpu.VMEM((E,), jnp.float32),
        pltpu.VMEM((PP,), jnp.float32),
        pltpu.VMEM((T,), jnp.float32),
        pltpu.VMEM((E,), jnp.float32),
        pltpu.SemaphoreType.DMA,
        pltpu.SemaphoreType.DMA,
        pltpu.SemaphoreType.DMA,
        pltpu.SemaphoreType.DMA,
    ],
)(_sc_body)


def kernel(decoder_hidden_state, alpha_graph_attention_top, all_embeddings, W):
    inter = _tc_matmul(decoder_hidden_state, W)
    emb_flat = all_embeddings.reshape(-1)
    atop_flat = alpha_graph_attention_top.reshape(-1)
    alpha_flat, c_part = _sc_call(emb_flat, inter, atop_flat)
    c = c_part.reshape(B, NW // B, E).sum(axis=1)
    alpha = alpha_flat.reshape(B, G, T)
    return (c, alpha)
